# baseline (speedup 1.0000x reference)
import functools

import jax
import jax.numpy as jnp
from jax import lax
from jax.experimental import pallas as pl
from jax.experimental.pallas import tpu as pltpu
from jax.experimental.pallas import tpu_sc as plsc

_BLOCK_K = 1024


def _argmax_body(q_ref, a_ref, noise_ref, idx_ref, qn_ref, bv_ref, bi_ref):
    j = pl.program_id(0)
    nb = pl.num_programs(0)
    b = q_ref.shape[0]

    @pl.when(j == 0)
    def _init():
        q = q_ref[...]
        qn_ref[...] = q * (1.0 / jnp.sqrt(jnp.sum(q * q, axis=1, keepdims=True)))
        bv_ref[...] = jnp.full((b,), -jnp.inf, dtype=jnp.float32)
        bi_ref[...] = jnp.zeros((b,), dtype=jnp.int32)

    a = a_ref[...]
    an = a * (1.0 / jnp.sqrt(jnp.sum(a * a, axis=1, keepdims=True)))
    m = lax.dot_general(qn_ref[...], an, (((1,), (1,)), ((), ())),
                        preferred_element_type=jnp.float32)
    d = jnp.abs((1.0 - m) - 0.5)
    g = d * jnp.float32(-(1.0 / jnp.float32(0.0001))) + noise_ref[...]

    lmax = jnp.max(g, axis=1)
    lidx = jnp.argmax(g, axis=1).astype(jnp.int32) + j * _BLOCK_K

    better = lmax > bv_ref[...]
    bi_ref[...] = jnp.where(better, lidx, bi_ref[...])
    bv_ref[...] = jnp.where(better, lmax, bv_ref[...])

    @pl.when(j == nb - 1)
    def _done():
        idx_ref[...] = bi_ref[...]


def _tc_argmax(id_vectors, anchors, gumbel_noise):
    b, d = id_vectors.shape
    k = anchors.shape[0]
    nb = k // _BLOCK_K
    return pl.pallas_call(
        _argmax_body,
        grid=(nb,),
        in_specs=[
            pl.BlockSpec((b, d), lambda j: (0, 0)),
            pl.BlockSpec((_BLOCK_K, d), lambda j: (j, 0)),
            pl.BlockSpec((b, _BLOCK_K), lambda j: (0, j)),
        ],
        out_specs=pl.BlockSpec((b,), lambda j: (0,)),
        out_shape=jax.ShapeDtypeStruct((b,), jnp.int32),
        scratch_shapes=[
            pltpu.VMEM((b, d), jnp.float32),
            pltpu.VMEM((b,), jnp.float32),
            pltpu.VMEM((b,), jnp.int32),
        ],
    )(id_vectors, anchors, gumbel_noise)


def _sc_gather(table, idx):
    info = plsc.get_sparse_core_info()
    nc, ns = info.num_cores, info.num_subcores
    nw = nc * ns
    b = idx.shape[0]
    d = table.shape[1]
    bpw = b // nw
    mesh = plsc.VectorSubcoreMesh(core_axis_name="c", subcore_axis_name="s")

    @functools.partial(
        pl.kernel, mesh=mesh,
        out_type=jax.ShapeDtypeStruct((b, d), jnp.float32),
        scratch_types=[
            pltpu.VMEM((bpw,), jnp.int32),
            pltpu.VMEM((bpw, d), jnp.float32),
            pltpu.SemaphoreType.DMA,
        ],
    )
    def gather_rows(table_hbm, idx_hbm, out_hbm, idx_v, rows_v, sem):
        wid = lax.axis_index("s") * nc + lax.axis_index("c")
        base = wid * bpw
        pltpu.sync_copy(idx_hbm.at[pl.ds(base, bpw)], idx_v)
        pltpu.async_copy(table_hbm.at[idx_v], rows_v, sem).wait()
        pltpu.sync_copy(rows_v, out_hbm.at[pl.ds(base, bpw)])

    return gather_rows(table, idx)


def kernel(id_vectors, anchors, gumbel_noise):
    idx = _tc_argmax(id_vectors, anchors, gumbel_noise)
    return _sc_gather(anchors, idx)


# speedup vs baseline: 1.2663x; 1.2663x over previous
import functools

import jax
import jax.numpy as jnp
from jax import lax
from jax.experimental import pallas as pl
from jax.experimental.pallas import tpu as pltpu
from jax.experimental.pallas import tpu_sc as plsc

_BLOCK_K = 4096


def _argmax_body(q_ref, a_ref, noise_ref, idx_ref, qn_ref, bv_ref, bi_ref):
    j = pl.program_id(0)
    nb = pl.num_programs(0)
    b = q_ref.shape[0]

    @pl.when(j == 0)
    def _init():
        q = q_ref[...]
        qn_ref[...] = q * (1.0 / jnp.sqrt(jnp.sum(q * q, axis=1, keepdims=True)))
        bv_ref[...] = jnp.full((b,), -jnp.inf, dtype=jnp.float32)
        bi_ref[...] = jnp.zeros((b,), dtype=jnp.int32)

    a = a_ref[...]
    an = a * (1.0 / jnp.sqrt(jnp.sum(a * a, axis=1, keepdims=True)))
    m = lax.dot_general(qn_ref[...], an, (((1,), (1,)), ((), ())),
                        preferred_element_type=jnp.float32)
    d = jnp.abs((1.0 - m) - 0.5)
    g = d * jnp.float32(-(1.0 / jnp.float32(0.0001))) + noise_ref[...]

    lmax = jnp.max(g, axis=1)
    lidx = jnp.argmax(g, axis=1).astype(jnp.int32) + j * _BLOCK_K

    better = lmax > bv_ref[...]
    bi_ref[...] = jnp.where(better, lidx, bi_ref[...])
    bv_ref[...] = jnp.where(better, lmax, bv_ref[...])

    @pl.when(j == nb - 1)
    def _done():
        idx_ref[...] = bi_ref[...]


def _tc_argmax(id_vectors, anchors, gumbel_noise):
    b, d = id_vectors.shape
    k = anchors.shape[0]
    nb = k // _BLOCK_K
    return pl.pallas_call(
        _argmax_body,
        grid=(nb,),
        in_specs=[
            pl.BlockSpec((b, d), lambda j: (0, 0)),
            pl.BlockSpec((_BLOCK_K, d), lambda j: (j, 0)),
            pl.BlockSpec((b, _BLOCK_K), lambda j: (0, j)),
        ],
        out_specs=pl.BlockSpec((b,), lambda j: (0,)),
        out_shape=jax.ShapeDtypeStruct((b,), jnp.int32),
        scratch_shapes=[
            pltpu.VMEM((b, d), jnp.float32),
            pltpu.VMEM((b,), jnp.float32),
            pltpu.VMEM((b,), jnp.int32),
        ],
    )(id_vectors, anchors, gumbel_noise)


def _sc_gather(table, idx):
    info = plsc.get_sparse_core_info()
    nc, ns = info.num_cores, info.num_subcores
    nw = nc * ns
    b = idx.shape[0]
    d = table.shape[1]
    bpw = b // nw
    mesh = plsc.VectorSubcoreMesh(core_axis_name="c", subcore_axis_name="s")

    @functools.partial(
        pl.kernel, mesh=mesh,
        out_type=jax.ShapeDtypeStruct((b, d), jnp.float32),
        scratch_types=[
            pltpu.VMEM((bpw,), jnp.int32),
            pltpu.VMEM((bpw, d), jnp.float32),
            pltpu.SemaphoreType.DMA,
        ],
    )
    def gather_rows(table_hbm, idx_hbm, out_hbm, idx_v, rows_v, sem):
        wid = lax.axis_index("s") * nc + lax.axis_index("c")
        base = wid * bpw
        pltpu.sync_copy(idx_hbm.at[pl.ds(base, bpw)], idx_v)
        pltpu.async_copy(table_hbm.at[idx_v], rows_v, sem).wait()
        pltpu.sync_copy(rows_v, out_hbm.at[pl.ds(base, bpw)])

    return gather_rows(table, idx)


def kernel(id_vectors, anchors, gumbel_noise):
    idx = _tc_argmax(id_vectors, anchors, gumbel_noise)
    return _sc_gather(anchors, idx)


# speedup vs baseline: 1.3313x; 1.0514x over previous
import functools

import jax
import jax.numpy as jnp
from jax import lax
from jax.experimental import pallas as pl
from jax.experimental.pallas import tpu as pltpu
from jax.experimental.pallas import tpu_sc as plsc

_BLOCK_K = 4096


def _argmax_body(q_ref, a_ref, noise_ref, idx_ref, qn_ref, bv_ref, bi_ref):
    j = pl.program_id(0)
    nb = pl.num_programs(0)
    b = q_ref.shape[0]

    scale = jnp.float32(1.0 / jnp.float32(0.0001))

    @pl.when(j == 0)
    def _init():
        q = q_ref[...]
        qn_ref[...] = q * (scale / jnp.sqrt(jnp.sum(q * q, axis=1, keepdims=True)))
        bv_ref[...] = jnp.full((b,), -jnp.inf, dtype=jnp.float32)
        bi_ref[...] = jnp.zeros((b,), dtype=jnp.int32)

    a = a_ref[...]
    an = a * (1.0 / jnp.sqrt(jnp.sum(a * a, axis=1, keepdims=True)))
    m = lax.dot_general(qn_ref[...], an, (((1,), (1,)), ((), ())),
                        preferred_element_type=jnp.float32)
    g = noise_ref[...] - jnp.abs(m - (0.5 * scale))

    lmax = jnp.max(g, axis=1)
    lidx = jnp.argmax(g, axis=1).astype(jnp.int32) + j * _BLOCK_K

    better = lmax > bv_ref[...]
    bi_ref[...] = jnp.where(better, lidx, bi_ref[...])
    bv_ref[...] = jnp.where(better, lmax, bv_ref[...])

    @pl.when(j == nb - 1)
    def _done():
        idx_ref[...] = bi_ref[...]


def _tc_argmax(id_vectors, anchors, gumbel_noise):
    b, d = id_vectors.shape
    k = anchors.shape[0]
    nb = k // _BLOCK_K
    return pl.pallas_call(
        _argmax_body,
        grid=(nb,),
        in_specs=[
            pl.BlockSpec((b, d), lambda j: (0, 0)),
            pl.BlockSpec((_BLOCK_K, d), lambda j: (j, 0)),
            pl.BlockSpec((b, _BLOCK_K), lambda j: (0, j)),
        ],
        out_specs=pl.BlockSpec((b,), lambda j: (0,)),
        out_shape=jax.ShapeDtypeStruct((b,), jnp.int32),
        scratch_shapes=[
            pltpu.VMEM((b, d), jnp.float32),
            pltpu.VMEM((b,), jnp.float32),
            pltpu.VMEM((b,), jnp.int32),
        ],
    )(id_vectors, anchors, gumbel_noise)


def _sc_gather(table, idx):
    info = plsc.get_sparse_core_info()
    nc, ns = info.num_cores, info.num_subcores
    nw = nc * ns
    b = idx.shape[0]
    d = table.shape[1]
    bpw = b // nw
    mesh = plsc.VectorSubcoreMesh(core_axis_name="c", subcore_axis_name="s")

    @functools.partial(
        pl.kernel, mesh=mesh,
        out_type=jax.ShapeDtypeStruct((b, d), jnp.float32),
        scratch_types=[
            pltpu.VMEM((bpw,), jnp.int32),
            pltpu.VMEM((bpw, d), jnp.float32),
            pltpu.SemaphoreType.DMA,
        ],
    )
    def gather_rows(table_hbm, idx_hbm, out_hbm, idx_v, rows_v, sem):
        wid = lax.axis_index("s") * nc + lax.axis_index("c")
        base = wid * bpw
        pltpu.sync_copy(idx_hbm.at[pl.ds(base, bpw)], idx_v)
        pltpu.async_copy(table_hbm.at[idx_v], rows_v, sem).wait()
        pltpu.sync_copy(rows_v, out_hbm.at[pl.ds(base, bpw)])

    return gather_rows(table, idx)


def kernel(id_vectors, anchors, gumbel_noise):
    idx = _tc_argmax(id_vectors, anchors, gumbel_noise)
    return _sc_gather(anchors, idx)
